# baseline (speedup 1.0000x reference)
import jax
import jax.numpy as jnp
from jax.experimental import pallas as pl

_B, _C, _H, _W, _E = 2, 384, 48, 48, 128
_N = _H * _W
_SIGMA = 0.05
_SCALE = float(_E) ** (-0.5)
_NEG = float(jnp.finfo(jnp.float32).min)
_LOG2E = 1.4426950408889634

_BLK_QKV = 768
_BLK_Q = 256


def _qkv_kernel(x_ref, w_ref, o_ref):
    o_ref[0] = jax.lax.dot_general(
        w_ref[...].astype(jnp.bfloat16), x_ref[0].astype(jnp.bfloat16),
        (((1,), (0,)), ((), ())),
        preferred_element_type=jnp.float32,
    )


def _attn_kernel(x_ref, q_ref, k_ref, v_ref, qa_ref, ka_ref, hi_ref, wp_ref,
                 o_ref):
    q = (q_ref[0] * (_SCALE * _LOG2E)).astype(jnp.bfloat16)
    k = k_ref[0].astype(jnp.bfloat16)
    v = v_ref[0].astype(jnp.bfloat16)

    s = jax.lax.dot_general(
        q, k, (((0,), (0,)), ((), ())),
        preferred_element_type=jnp.float32,
    )
    s = s + jax.lax.dot_general(
        qa_ref[...], ka_ref[0], (((0,), (0,)), ((), ())),
        preferred_element_type=jnp.float32,
        precision=jax.lax.Precision.HIGHEST,
    )

    m = jnp.max(s, axis=1, keepdims=True)
    p = jnp.exp2(s - m)
    l = jnp.sum(p, axis=1, keepdims=True)

    agg_t = jax.lax.dot_general(
        v, p.astype(jnp.bfloat16), (((1,), (1,)), ((), ())),
        preferred_element_type=jnp.float32,
    )
    agg_t = agg_t * (1.0 / l).reshape(1, _BLK_Q)
    delta_t = jax.lax.dot_general(
        wp_ref[...].astype(jnp.bfloat16), agg_t.astype(jnp.bfloat16),
        (((1,), (0,)), ((), ())),
        preferred_element_type=jnp.float32,
    )
    o_ref[0] = x_ref[0] + jnp.where(hi_ref[0] > 0, delta_t, 0.0)


@jax.jit
def kernel(feat, mask_hi, Wq, Wk, Wv, Wp):
    x = feat.reshape(_B, _C, _N)
    wcat = jnp.concatenate([Wq, Wk, Wv], axis=0)
    hi = mask_hi.reshape(_B, 1, _N).astype(jnp.float32)

    idx = jnp.arange(_N, dtype=jnp.int32)
    py = (idx // _W).astype(jnp.float32) * (1.0 / (_H - 1))
    px = (idx % _W).astype(jnp.float32) * (1.0 / (_W - 1))
    cl = 200.0 * _LOG2E
    zero = jnp.zeros((_N,), jnp.float32)
    ones = jnp.ones((_N,), jnp.float32)
    qaug = jnp.stack([
        2.0 * cl * py, 2.0 * cl * px, ones, -cl * (py * py + px * px),
        zero, zero, zero, zero], axis=0)
    k2 = -cl * (py * py + px * px)[None, :] + hi[:, 0, :] * _NEG
    kaug = jnp.stack([
        jnp.broadcast_to(py, (_B, _N)), jnp.broadcast_to(px, (_B, _N)),
        k2, jnp.broadcast_to(ones, (_B, _N)),
        jnp.broadcast_to(zero, (_B, _N)), jnp.broadcast_to(zero, (_B, _N)),
        jnp.broadcast_to(zero, (_B, _N)), jnp.broadcast_to(zero, (_B, _N)),
    ], axis=1)

    qkv = pl.pallas_call(
        _qkv_kernel,
        grid=(_B, _N // _BLK_QKV),
        in_specs=[
            pl.BlockSpec((1, _C, _BLK_QKV), lambda b, n: (b, 0, n)),
            pl.BlockSpec((3 * _E, _C), lambda b, n: (0, 0)),
        ],
        out_specs=pl.BlockSpec((1, 3 * _E, _BLK_QKV), lambda b, n: (b, 0, n)),
        out_shape=jax.ShapeDtypeStruct((_B, 3 * _E, _N), jnp.float32),
    )(x, wcat)

    out = pl.pallas_call(
        _attn_kernel,
        grid=(_B, _N // _BLK_Q),
        in_specs=[
            pl.BlockSpec((1, _C, _BLK_Q), lambda b, q: (b, 0, q)),
            pl.BlockSpec((1, _E, _BLK_Q), lambda b, q: (b, 0, q)),
            pl.BlockSpec((1, _E, _N), lambda b, q: (b, 1, 0)),
            pl.BlockSpec((1, _E, _N), lambda b, q: (b, 2, 0)),
            pl.BlockSpec((8, _BLK_Q), lambda b, q: (0, q)),
            pl.BlockSpec((1, 8, _N), lambda b, q: (b, 0, 0)),
            pl.BlockSpec((1, 1, _BLK_Q), lambda b, q: (b, 0, q)),
            pl.BlockSpec((_C, _E), lambda b, q: (0, 0)),
        ],
        out_specs=pl.BlockSpec((1, _C, _BLK_Q), lambda b, q: (b, 0, q)),
        out_shape=jax.ShapeDtypeStruct((_B, _C, _N), jnp.float32),
    )(x, qkv, qkv, qkv, qaug, kaug, hi, Wp)

    return out.reshape(_B, _C, _H, _W)


# speedup vs baseline: 1.3734x; 1.3734x over previous
import jax
import jax.numpy as jnp
from jax.experimental import pallas as pl

_B, _C, _H, _W, _E = 2, 384, 48, 48, 128
_N = _H * _W
_SIGMA = 0.05
_SCALE = float(_E) ** (-0.5)
_NEG = float(jnp.finfo(jnp.float32).min)
_LOG2E = 1.4426950408889634
_CB = 200.0 * _LOG2E / ((_H - 1) * (_H - 1))

_BLK_QKV = 768
_BLK_Q = 256


def _qkv_kernel(x_ref, w_ref, o_ref):
    o_ref[0] = jax.lax.dot_general(
        w_ref[...].astype(jnp.bfloat16), x_ref[0].astype(jnp.bfloat16),
        (((1,), (0,)), ((), ())),
        preferred_element_type=jnp.float32,
    )


def _attn_kernel(x_ref, q_ref, k_ref, v_ref, qi_ref, ki_ref, col_ref, hi_ref,
                 wp_ref, o_ref):
    q = (q_ref[0] * (_SCALE * _LOG2E)).astype(jnp.bfloat16)
    k = k_ref[0].astype(jnp.bfloat16)
    v = v_ref[0].astype(jnp.bfloat16)

    s = jax.lax.dot_general(
        q, k, (((0,), (0,)), ((), ())),
        preferred_element_type=jnp.float32,
    )
    cross = jax.lax.dot_general(
        qi_ref[...], ki_ref[...], (((0,), (0,)), ((), ())),
        preferred_element_type=jnp.float32,
    )
    s = s + (cross * (2.0 * _CB) + col_ref[0])

    m = jnp.max(s, axis=1, keepdims=True)
    p = jnp.exp2(s - m)
    l = jnp.sum(p, axis=1, keepdims=True)

    agg_t = jax.lax.dot_general(
        v, p.astype(jnp.bfloat16), (((1,), (1,)), ((), ())),
        preferred_element_type=jnp.float32,
    )
    agg_t = agg_t * (1.0 / l).reshape(1, _BLK_Q)
    delta_t = jax.lax.dot_general(
        wp_ref[...].astype(jnp.bfloat16), agg_t.astype(jnp.bfloat16),
        (((1,), (0,)), ((), ())),
        preferred_element_type=jnp.float32,
    )
    o_ref[0] = x_ref[0] + jnp.where(hi_ref[0] > 0, delta_t, 0.0)


@jax.jit
def kernel(feat, mask_hi, Wq, Wk, Wv, Wp):
    x = feat.reshape(_B, _C, _N)
    wcat = jnp.concatenate([Wq, Wk, Wv], axis=0)
    hi = mask_hi.reshape(_B, 1, _N).astype(jnp.float32)

    idx = jnp.arange(_N, dtype=jnp.int32)
    gi = (idx // _W).astype(jnp.float32)
    gj = (idx % _W).astype(jnp.float32)
    zero = jnp.zeros((_N,), jnp.float32)
    coords = jnp.stack([gi, gj, zero, zero, zero, zero, zero, zero],
                       axis=0).astype(jnp.bfloat16)
    col = (-_CB * (gi * gi + gj * gj))[None, None, :] + hi * _NEG

    qkv = pl.pallas_call(
        _qkv_kernel,
        grid=(_B, _N // _BLK_QKV),
        in_specs=[
            pl.BlockSpec((1, _C, _BLK_QKV), lambda b, n: (b, 0, n)),
            pl.BlockSpec((3 * _E, _C), lambda b, n: (0, 0)),
        ],
        out_specs=pl.BlockSpec((1, 3 * _E, _BLK_QKV), lambda b, n: (b, 0, n)),
        out_shape=jax.ShapeDtypeStruct((_B, 3 * _E, _N), jnp.float32),
    )(x, wcat)

    out = pl.pallas_call(
        _attn_kernel,
        grid=(_B, _N // _BLK_Q),
        in_specs=[
            pl.BlockSpec((1, _C, _BLK_Q), lambda b, q: (b, 0, q)),
            pl.BlockSpec((1, _E, _BLK_Q), lambda b, q: (b, 0, q)),
            pl.BlockSpec((1, _E, _N), lambda b, q: (b, 1, 0)),
            pl.BlockSpec((1, _E, _N), lambda b, q: (b, 2, 0)),
            pl.BlockSpec((8, _BLK_Q), lambda b, q: (0, q)),
            pl.BlockSpec((8, _N), lambda b, q: (0, 0)),
            pl.BlockSpec((1, 1, _N), lambda b, q: (b, 0, 0)),
            pl.BlockSpec((1, 1, _BLK_Q), lambda b, q: (b, 0, q)),
            pl.BlockSpec((_C, _E), lambda b, q: (0, 0)),
        ],
        out_specs=pl.BlockSpec((1, _C, _BLK_Q), lambda b, q: (b, 0, q)),
        out_shape=jax.ShapeDtypeStruct((_B, _C, _N), jnp.float32),
    )(x, qkv, qkv, qkv, coords, coords, col, hi, Wp)

    return out.reshape(_B, _C, _H, _W)
